# Initial kernel scaffold; baseline (speedup 1.0000x reference)
import jax
import jax.numpy as jnp
from jax.experimental import pallas as pl


def kernel(residual, codebooks):
    raise NotImplementedError("write your pallas kernel here")


# speedup vs baseline: 1.0617x; 1.0617x over previous
import functools

import jax
import jax.numpy as jnp
from jax import lax
from jax.experimental import pallas as pl
from jax.experimental.pallas import tpu as pltpu
from jax.experimental.pallas import tpu_sc as plsc

_R = 256
_C = 2048
_CH = 128
_NC = 2
_NS = 16


def _argmin_body(x2_ref, res_ref, cbt_ref, e2_ref, idx_ref, *, n_k, bounds):
    res = res_ref[...]
    x2 = x2_ref[...]
    nch = len(bounds)
    ch_min = [jnp.full((_R, 1), jnp.inf, jnp.float32) for _ in range(nch)]
    ch_arg = [jnp.zeros((_R, 1), jnp.int32) for _ in range(nch)]
    for j in range(n_k // _C):
        t = lax.dot_general(res, cbt_ref[:, j * _C:(j + 1) * _C],
                            (((1,), (0,)), ((), ())),
                            preferred_element_type=jnp.float32)
        sc = (x2 + e2_ref[:, j * _C:(j + 1) * _C]) + t
        ii = lax.broadcasted_iota(jnp.int32, (_R, _C), 1) + j * _C
        lo_p, hi_p = j * _C, (j + 1) * _C
        for i, (lo, hi) in enumerate(bounds):
            if hi <= lo_p or lo >= hi_p:
                continue
            if lo <= lo_p and hi >= hi_p:
                scm, iim = sc, ii
            else:
                msk = (ii >= lo) & (ii < hi)
                scm = jnp.where(msk, sc, jnp.inf)
                iim = jnp.where(msk, ii, n_k)
            m = jnp.min(scm, axis=1, keepdims=True)
            a = jnp.min(jnp.where(scm == m, iim, n_k), axis=1, keepdims=True)
            upd = m < ch_min[i]
            ch_arg[i] = jnp.where(upd, a, ch_arg[i])
            ch_min[i] = jnp.where(upd, m, ch_min[i])
    best = jnp.full((_R, 1), jnp.inf, jnp.float32)
    barg = jnp.zeros((_R, 1), jnp.int32)
    for i in range(nch):
        upd = ch_min[i] < best
        barg = jnp.where(upd, ch_arg[i], barg)
        best = jnp.where(upd,
                         ch_min[i].astype(jnp.bfloat16).astype(jnp.float32),
                         best)
    idx_ref[...] = barg


def _argmin_level(res_bf, x2, cbt_bf, e2, bounds):
    b, d = res_bf.shape
    n_k = cbt_bf.shape[1]
    return pl.pallas_call(
        functools.partial(_argmin_body, n_k=n_k, bounds=bounds),
        grid=(b // _R,),
        in_specs=[
            pl.BlockSpec((_R, 1), lambda i: (i, 0)),
            pl.BlockSpec((_R, d), lambda i: (i, 0)),
            pl.BlockSpec((d, n_k), lambda i: (0, 0)),
            pl.BlockSpec((1, n_k), lambda i: (0, 0)),
        ],
        out_specs=pl.BlockSpec((_R, 1), lambda i: (i, 0)),
        out_shape=jax.ShapeDtypeStruct((b, 1), jnp.int32),
    )(x2, res_bf, cbt_bf, e2)


def _gather_level(cb, idx):
    b = idx.shape[0]
    k, d = cb.shape
    nw = _NC * _NS
    bpw = b // nw
    chunks = bpw // _CH
    mesh = plsc.VectorSubcoreMesh(core_axis_name="c", subcore_axis_name="s")

    @functools.partial(
        pl.kernel, mesh=mesh,
        out_type=jax.ShapeDtypeStruct((b, d), jnp.float32),
        scratch_types=[
            pltpu.VMEM((_CH,), jnp.int32),
            pltpu.VMEM((_CH, d), jnp.float32),
            pltpu.SemaphoreType.DMA,
        ],
    )
    def gk(cb_hbm, idx_hbm, out_hbm, idx_v, rows_v, sem):
        wid = lax.axis_index("s") * _NC + lax.axis_index("c")
        base = pl.multiple_of(wid * bpw, _CH)
        for c in range(chunks):
            off = pl.multiple_of(base + c * _CH, _CH)
            pltpu.sync_copy(idx_hbm.at[pl.ds(off, _CH)], idx_v)
            pltpu.async_copy(cb_hbm.at[idx_v], rows_v, sem).wait()
            pltpu.sync_copy(rows_v, out_hbm.at[pl.ds(off, _CH)])

    return gk(cb, idx)


def _chunk_bounds(l, n_k):
    w = 2736
    return [(lo, min(n_k, lo + w)) for lo in range(0, n_k, w)]


def kernel(residual, codebooks):
    n_l = codebooks.shape[0]
    n_k = codebooks.shape[1]
    res = residual
    qsum = jnp.zeros_like(residual)
    codes = []
    for l in range(n_l):
        emb = codebooks[l]
        cbt_bf = (-2.0 * emb.astype(jnp.bfloat16).astype(jnp.float32)
                  ).astype(jnp.bfloat16).T
        e2 = jnp.sum(emb ** 2, axis=1)
        x2 = jnp.sum(res ** 2, axis=1, keepdims=True)
        idx2d = _argmin_level(res.astype(jnp.bfloat16), x2, cbt_bf, e2[None, :],
                              _chunk_bounds(l, n_k))
        idx = idx2d[:, 0]
        q = _gather_level(emb, idx)
        codes.append(idx)
        qsum = qsum + q
        res = res - q
    return (qsum, jnp.stack(codes, axis=1))
